# Initial kernel scaffold; baseline (speedup 1.0000x reference)
import jax
import jax.numpy as jnp
from jax.experimental import pallas as pl


def kernel(node_type, node_tactic_id, state_lm_id, batch, edge_index, state_lm_bank, params):
    raise NotImplementedError("write your pallas kernel here")


# speedup vs baseline: 3.8604x; 3.8604x over previous
import functools

import jax
import jax.numpy as jnp
from jax import lax
from jax.experimental import pallas as pl
from jax.experimental.pallas import tpu as pltpu
from jax.experimental.pallas import tpu_sc as plsc

N = 10000
E = 320000
B = 256
NUM_TACTICS = 512
LM_DIM = 768
STATE_DIM = 128
HID = 512

NC = 2
NS = 16
E_PAD = 327680
EB = 1024
NBLK = E_PAD // EB
N_ACC = 10112
ROWS_PER_TILE = N_ACC // NS
CW = 64

@functools.lru_cache(maxsize=None)
def _mesh():
    return plsc.VectorSubcoreMesh(
        core_axis_name="c", subcore_axis_name="s",
        num_cores=NC, num_subcores=NS)


f32 = jnp.float32


@functools.lru_cache(maxsize=None)
def _make_segsum(C):
    assert C % NC == 0
    chunks_per_core = C // NC

    @functools.partial(
        pl.kernel,
        out_type=jax.ShapeDtypeStruct((C * N_ACC, CW), f32),
        mesh=_mesh(),
        compiler_params=pltpu.CompilerParams(use_tc_tiling_on_sc=False),
        scratch_types=[
            pltpu.VMEM_SHARED((N_ACC, CW), f32),
            pltpu.VMEM((8, 128), jnp.int32),
            pltpu.VMEM((8, 128), jnp.int32),
            pltpu.VMEM((128, CW), f32),
            pltpu.VMEM((ROWS_PER_TILE, CW), f32),
        ],
    )
    def segsum(x_hbm, src_hbm, dst_hbm, out_hbm, acc, sidx, didx, rows, zbuf):
        k = lax.axis_index("c")
        s = lax.axis_index("s")
        blk0 = s * (NBLK // NS)
        nblk = NBLK // NS

        @pl.loop(0, ROWS_PER_TILE)
        def _zero(r):
            for cc in range(CW // 16):
                zbuf[r, pl.ds(cc * 16, 16)] = jnp.zeros((16,), f32)

        for ci in range(chunks_per_core):
            chunk = k * chunks_per_core + ci
            out_base = chunk * N_ACC

            if ci > 0:
                plsc.subcore_barrier()
            pltpu.sync_copy(zbuf,
                            acc.at[pl.ds(s * ROWS_PER_TILE, ROWS_PER_TILE)])
            plsc.subcore_barrier()

            src_base = chunk * NBLK + blk0

            @pl.loop(0, nblk)
            def _edges(b):
                pltpu.sync_copy(src_hbm.at[src_base + b], sidx)
                pltpu.sync_copy(dst_hbm.at[blk0 + b], didx)
                for j in range(8):
                    pltpu.sync_copy(x_hbm.at[sidx.at[j]], rows)
                    pltpu.sync_copy(rows, acc.at[didx.at[j]], add=True)

            plsc.subcore_barrier()
            pltpu.sync_copy(
                acc.at[pl.ds(s * ROWS_PER_TILE, ROWS_PER_TILE)],
                out_hbm.at[pl.ds(out_base + s * ROWS_PER_TILE, ROWS_PER_TILE)])

    return segsum


N_LM_PAD = 10240
LM_PER_TILE = N_LM_PAD // (NC * NS)
LM_SUB = 64


@functools.lru_cache(maxsize=None)
def _make_lm_gather():
    @functools.partial(
        pl.kernel,
        out_type=jax.ShapeDtypeStruct((N_LM_PAD, LM_DIM), f32),
        mesh=_mesh(),
        scratch_types=[
            pltpu.VMEM((LM_PER_TILE,), jnp.int32),
            pltpu.VMEM((LM_SUB, LM_DIM), f32),
        ],
    )
    def lm_gather(bank_hbm, idx_hbm, out_hbm, idxv, rows):
        k = lax.axis_index("c")
        s = lax.axis_index("s")
        base = (k * NS + s) * LM_PER_TILE
        pltpu.sync_copy(idx_hbm.at[pl.ds(base, LM_PER_TILE)], idxv)
        for b in range(LM_PER_TILE // LM_SUB):
            pltpu.sync_copy(bank_hbm.at[idxv.at[pl.ds(b * LM_SUB, LM_SUB)]],
                            rows)
            pltpu.sync_copy(rows, out_hbm.at[pl.ds(base + b * LM_SUB, LM_SUB)])

    return lm_gather


BLK = 1000
GRID = N // BLK


def _tc1_body(nt_ref, sh_ref, temb_ref, tacp_ref, wr_ref, x0c_ref, y0r_ref):
    nt = nt_ref[...]
    sh = sh_ref[...]
    t_type = jnp.zeros((BLK, 32), f32)
    for kk in range(3):
        t_type = t_type + (nt == kk).astype(f32) * temb_ref[pl.ds(kk, 1), :]
    onehot = (sh == lax.broadcasted_iota(jnp.int32, (1, 640), 1)).astype(f32)
    t_tac = jnp.dot(onehot, tacp_ref[...], preferred_element_type=f32)
    x0p = jnp.concatenate(
        [t_type, t_tac, jnp.ones((BLK, 1), f32), jnp.zeros((BLK, 31), f32)],
        axis=1)
    x0c_ref[0, :, :] = x0p[:, :CW]
    x0c_ref[1, :, :] = x0p[:, CW:]
    y0r_ref[...] = jnp.dot(x0p, wr_ref[...], preferred_element_type=f32)


def _tc2_body(p_ref, y0r_ref, wl_ref, bl_ref, x1c_ref, invd_ref):
    p = p_ref[...]
    ssum = jnp.concatenate([p[0], p[1]], axis=1)
    deg = ssum[:, 96:97]
    invd = 1.0 / jnp.maximum(deg, 1.0)
    mean1 = ssum * invd
    x1 = jnp.maximum(
        jnp.dot(mean1, wl_ref[...], preferred_element_type=f32)
        + bl_ref[...] + y0r_ref[...], 0.0)
    for c in range(8):
        x1c_ref[c, :, :] = x1[:, c * CW:(c + 1) * CW]
    invd_ref[...] = invd


def _tc3_body(a2_ref, x1c_ref, invd_ref, batch_ref, wl_ref, bl_ref, wr_ref,
              gw1_ref, gb1_ref, gw2_ref, gb2_ref, x2_ref, g_ref, gmax_ref):
    i = pl.program_id(0)
    invd = invd_ref[...]
    acc = jnp.broadcast_to(bl_ref[...], (BLK, HID))
    for c in range(8):
        acc = acc + jnp.dot(a2_ref[c] * invd, wl_ref[pl.ds(c * CW, CW), :],
                            preferred_element_type=f32)
        acc = acc + jnp.dot(x1c_ref[c], wr_ref[pl.ds(c * CW, CW), :],
                            preferred_element_type=f32)
    x2 = jnp.maximum(acc, 0.0)
    gh = jnp.maximum(
        jnp.dot(x2, gw1_ref[...], preferred_element_type=f32) + gb1_ref[...],
        0.0)
    g = jnp.dot(gh, gw2_ref[...], preferred_element_type=f32) + gb2_ref[...]
    x2_ref[...] = x2
    g_ref[...] = g
    bm = batch_ref[...] == lax.broadcasted_iota(jnp.int32, (1, B), 1)
    cand = jnp.where(bm, g, -1e30)
    blockmax = jnp.max(cand, axis=0, keepdims=True)

    @pl.when(i == 0)
    def _():
        gmax_ref[...] = jnp.full((1, B), -1e30, f32)

    gmax_ref[...] = jnp.maximum(gmax_ref[...], blockmax)


def _tc4_body(x2_ref, g_ref, batch_ref, gmax_ref, lm_ref, sid_ref,
              sw_ref, sb_ref, lng_ref, lnb_ref,
              w1a_ref, w1b_ref, b1_ref, w2_ref, b2_ref,
              out_ref, S_ref, d_ref, Hs_ref, cnt_ref):
    i = pl.program_id(0)

    @pl.when(i == 0)
    def _():
        S_ref[...] = jnp.zeros((B, HID), f32)
        d_ref[...] = jnp.zeros((B, 1), f32)
        Hs_ref[...] = jnp.zeros((B, STATE_DIM), f32)
        cnt_ref[...] = jnp.zeros((B, 1), f32)

    P = (batch_ref[...] == lax.broadcasted_iota(jnp.int32, (1, B), 1)
         ).astype(f32)
    gmaxsel = jnp.sum(P * gmax_ref[...], axis=1, keepdims=True)
    ex = jnp.exp(g_ref[...] - gmaxsel)
    dn = (((0,), (0,)), ((), ()))
    S_ref[...] = S_ref[...] + lax.dot_general(
        P, ex * x2_ref[...], dn, preferred_element_type=f32)
    d_ref[...] = d_ref[...] + lax.dot_general(
        P, ex, dn, preferred_element_type=f32)

    hb = jnp.maximum(
        jnp.dot(lm_ref[...], sw_ref[...], preferred_element_type=f32)
        + sb_ref[...], 0.0)
    mu = jnp.mean(hb, axis=1, keepdims=True)
    var = jnp.mean((hb - mu) * (hb - mu), axis=1, keepdims=True)
    h = (hb - mu) / jnp.sqrt(var + 1e-5) * lng_ref[...] + lnb_ref[...]
    mask = (sid_ref[...] >= 0).astype(f32)
    h = h * mask
    Hs_ref[...] = Hs_ref[...] + lax.dot_general(
        P, h, dn, preferred_element_type=f32)
    cnt_ref[...] = cnt_ref[...] + lax.dot_general(
        P, mask, dn, preferred_element_type=f32)

    @pl.when(i == GRID - 1)
    def _():
        graph_struct = S_ref[...] / (d_ref[...] + 1e-16)
        state_sem = Hs_ref[...] / (cnt_ref[...] + 1e-6)
        hcls = jnp.maximum(
            jnp.dot(graph_struct, w1a_ref[...], preferred_element_type=f32)
            + jnp.dot(state_sem, w1b_ref[...], preferred_element_type=f32)
            + b1_ref[...], 0.0)
        out_ref[...] = (jnp.dot(hcls, w2_ref[...], preferred_element_type=f32)
                        + b2_ref[...])


def _full(shape):
    return pl.BlockSpec(shape, lambda i: (0,) * len(shape))


def kernel(node_type, node_tactic_id, state_lm_id, batch, edge_index,
           state_lm_bank, params):
    p = params
    src = edge_index[0]
    dst = edge_index[1]
    npad = E_PAD - E
    pad_src = jnp.arange(npad, dtype=jnp.int32) % N
    pad_dst = N + (jnp.arange(npad, dtype=jnp.int32) % (N_ACC - N))
    src_p = jnp.concatenate([src, pad_src])
    dst_p = jnp.concatenate([dst, pad_dst])
    dstb = dst_p.reshape(NBLK, 8, 128)
    src2 = (src_p[None, :]
            + (jnp.arange(2, dtype=jnp.int32) * N)[:, None]).reshape(
                2 * NBLK, 8, 128)
    src8 = (src_p[None, :]
            + (jnp.arange(8, dtype=jnp.int32) * N)[:, None]).reshape(
                8 * NBLK, 8, 128)

    nt2 = node_type[:, None]
    sh2 = jnp.clip(node_tactic_id + 1, 0, NUM_TACTICS)[:, None]
    batch2 = batch[:, None]
    sid2 = state_lm_id[:, None]

    temb = jnp.zeros((8, 32), f32).at[:3].set(p["type_emb"])
    tacp = jnp.zeros((640, 64), f32).at[:NUM_TACTICS + 1].set(p["tactic_emb"])
    wr_pad = jnp.zeros((128, HID), f32).at[:96].set(p["c1_Wr"])
    wl_pad = jnp.zeros((128, HID), f32).at[:96].set(p["c1_Wl"])
    bl1 = p["c1_bl"][None, :]
    bl2 = p["c2_bl"][None, :]
    gb1 = p["gate_b1"][None, :]
    gb2 = p["gate_b2"][None, :]
    sb = p["state_b"][None, :]
    lng = p["state_ln_g"][None, :]
    lnb = p["state_ln_b"][None, :]
    w1a = p["cls_W1"][:HID]
    w1b = p["cls_W1"][HID:]
    b1 = p["cls_b1"][None, :]
    b2 = p["cls_b2"][None, :]

    lm_idx = jnp.concatenate([
        jnp.clip(state_lm_id, 0),
        jnp.arange(N_LM_PAD - N, dtype=jnp.int32) % 17])

    x0c, y0r = pl.pallas_call(
        _tc1_body,
        grid=(GRID,),
        in_specs=[
            pl.BlockSpec((BLK, 1), lambda i: (i, 0)),
            pl.BlockSpec((BLK, 1), lambda i: (i, 0)),
            _full((8, 32)),
            _full((640, 64)),
            _full((128, HID)),
        ],
        out_specs=[
            pl.BlockSpec((2, BLK, CW), lambda i: (0, i, 0)),
            pl.BlockSpec((BLK, HID), lambda i: (i, 0)),
        ],
        out_shape=[
            jax.ShapeDtypeStruct((2, N, CW), f32),
            jax.ShapeDtypeStruct((N, HID), f32),
        ],
    )(nt2, sh2, temb, tacp, wr_pad)

    agg1 = _make_segsum(2)(x0c.reshape(2 * N, CW), src2,
                           dstb).reshape(2, N_ACC, CW)

    lm = _make_lm_gather()(state_lm_bank, lm_idx)

    x1c, invd = pl.pallas_call(
        _tc2_body,
        grid=(GRID,),
        in_specs=[
            pl.BlockSpec((2, BLK, CW), lambda i: (0, i, 0)),
            pl.BlockSpec((BLK, HID), lambda i: (i, 0)),
            _full((128, HID)),
            _full((1, HID)),
        ],
        out_specs=[
            pl.BlockSpec((8, BLK, CW), lambda i: (0, i, 0)),
            pl.BlockSpec((BLK, 1), lambda i: (i, 0)),
        ],
        out_shape=[
            jax.ShapeDtypeStruct((8, N, CW), f32),
            jax.ShapeDtypeStruct((N, 1), f32),
        ],
    )(agg1, y0r, wl_pad, bl1)

    agg2 = _make_segsum(8)(x1c.reshape(8 * N, CW), src8,
                           dstb).reshape(8, N_ACC, CW)

    x2, g, gmax = pl.pallas_call(
        _tc3_body,
        grid=(GRID,),
        in_specs=[
            pl.BlockSpec((8, BLK, CW), lambda i: (0, i, 0)),
            pl.BlockSpec((8, BLK, CW), lambda i: (0, i, 0)),
            pl.BlockSpec((BLK, 1), lambda i: (i, 0)),
            pl.BlockSpec((BLK, 1), lambda i: (i, 0)),
            _full((HID, HID)),
            _full((1, HID)),
            _full((HID, HID)),
            _full((HID, HID)),
            _full((1, HID)),
            _full((HID, 1)),
            _full((1, 1)),
        ],
        out_specs=[
            pl.BlockSpec((BLK, HID), lambda i: (i, 0)),
            pl.BlockSpec((BLK, 1), lambda i: (i, 0)),
            pl.BlockSpec((1, B), lambda i: (0, 0)),
        ],
        out_shape=[
            jax.ShapeDtypeStruct((N, HID), f32),
            jax.ShapeDtypeStruct((N, 1), f32),
            jax.ShapeDtypeStruct((1, B), f32),
        ],
    )(agg2, x1c, invd, batch2, p["c2_Wl"], bl2, p["c2_Wr"],
      p["gate_W1"], gb1, p["gate_W2"], gb2)

    logits = pl.pallas_call(
        _tc4_body,
        grid=(GRID,),
        in_specs=[
            pl.BlockSpec((BLK, HID), lambda i: (i, 0)),
            pl.BlockSpec((BLK, 1), lambda i: (i, 0)),
            pl.BlockSpec((BLK, 1), lambda i: (i, 0)),
            _full((1, B)),
            pl.BlockSpec((BLK, LM_DIM), lambda i: (i, 0)),
            pl.BlockSpec((BLK, 1), lambda i: (i, 0)),
            _full((LM_DIM, STATE_DIM)),
            _full((1, STATE_DIM)),
            _full((1, STATE_DIM)),
            _full((1, STATE_DIM)),
            _full((HID, HID)),
            _full((STATE_DIM, HID)),
            _full((1, HID)),
            _full((HID, HID)),
            _full((1, HID)),
        ],
        out_specs=pl.BlockSpec((B, HID), lambda i: (0, 0)),
        out_shape=jax.ShapeDtypeStruct((B, HID), f32),
        scratch_shapes=[
            pltpu.VMEM((B, HID), f32),
            pltpu.VMEM((B, 1), f32),
            pltpu.VMEM((B, STATE_DIM), f32),
            pltpu.VMEM((B, 1), f32),
        ],
    )(x2, g, batch2, gmax, lm, sid2,
      p["state_W"], sb, lng, lnb, w1a, w1b, b1, p["cls_W2"], b2)

    return logits


# speedup vs baseline: 5.7054x; 1.4779x over previous
import functools

import jax
import jax.numpy as jnp
from jax import lax
from jax.experimental import pallas as pl
from jax.experimental.pallas import tpu as pltpu
from jax.experimental.pallas import tpu_sc as plsc

N = 10000
E = 320000
B = 256
NUM_TACTICS = 512
LM_DIM = 768
STATE_DIM = 128
HID = 512

NC = 2
NS = 16
E_PAD = 327680
EB = 1024
NBLK = E_PAD // EB
N_ACC = 10112
ROWS_PER_TILE = N_ACC // NS
CW = 64

@functools.lru_cache(maxsize=None)
def _mesh():
    return plsc.VectorSubcoreMesh(
        core_axis_name="c", subcore_axis_name="s",
        num_cores=NC, num_subcores=NS)


f32 = jnp.float32


@functools.lru_cache(maxsize=None)
def _make_segsum(C):
    assert C % NC == 0
    chunks_per_core = C // NC

    @functools.partial(
        pl.kernel,
        out_type=jax.ShapeDtypeStruct((C * N_ACC, CW), f32),
        mesh=_mesh(),
        compiler_params=pltpu.CompilerParams(use_tc_tiling_on_sc=False),
        scratch_types=[
            pltpu.VMEM_SHARED((N_ACC, CW), f32),
            pltpu.VMEM((2, EB // 2), jnp.int32),
            pltpu.VMEM((2, EB // 2), jnp.int32),
            pltpu.VMEM((EB // 2, CW), f32),
            pltpu.VMEM((EB // 2, CW), f32),
            pltpu.VMEM((ROWS_PER_TILE // 4, CW), f32),
            pltpu.SemaphoreType.DMA,
            pltpu.SemaphoreType.DMA,
        ],
    )
    def segsum(x_hbm, src_hbm, dst_hbm, out_hbm, acc, sidx, didx,
               rowsA, rowsB, zbuf, semA, semB):
        k = lax.axis_index("c")
        s = lax.axis_index("s")
        blk0 = s * (NBLK // NS)
        nblk = NBLK // NS

        @pl.loop(0, ROWS_PER_TILE // 4)
        def _zero(r):
            for cc in range(CW // 16):
                zbuf[r, pl.ds(cc * 16, 16)] = jnp.zeros((16,), f32)

        for ci in range(chunks_per_core):
            chunk = k * chunks_per_core + ci
            out_base = chunk * N_ACC

            if ci > 0:
                plsc.subcore_barrier()
            for q in range(4):
                pltpu.sync_copy(
                    zbuf,
                    acc.at[pl.ds(s * ROWS_PER_TILE + q * (ROWS_PER_TILE // 4),
                                 ROWS_PER_TILE // 4)])
            plsc.subcore_barrier()

            src_base = chunk * NBLK + blk0

            @pl.loop(0, nblk)
            def _edges(b):
                pltpu.sync_copy(src_hbm.at[src_base + b], sidx)
                pltpu.sync_copy(dst_hbm.at[blk0 + b], didx)
                ga = pltpu.async_copy(x_hbm.at[sidx.at[0]], rowsA, semA)
                ga.wait()
                gb = pltpu.async_copy(x_hbm.at[sidx.at[1]], rowsB, semB)
                sa = pltpu.async_copy(rowsA, acc.at[didx.at[0]],
                                      semA, add=True)
                gb.wait()
                sa.wait()
                sb = pltpu.async_copy(rowsB, acc.at[didx.at[1]],
                                      semB, add=True)
                sb.wait()

            plsc.subcore_barrier()
            pltpu.sync_copy(
                acc.at[pl.ds(s * ROWS_PER_TILE, ROWS_PER_TILE)],
                out_hbm.at[pl.ds(out_base + s * ROWS_PER_TILE, ROWS_PER_TILE)])

    return segsum


N_LM_PAD = 10240
LM_PER_TILE = N_LM_PAD // (NC * NS)
LM_SUB = 64


@functools.lru_cache(maxsize=None)
def _make_lm_gather():
    @functools.partial(
        pl.kernel,
        out_type=jax.ShapeDtypeStruct((N_LM_PAD, LM_DIM), f32),
        mesh=_mesh(),
        scratch_types=[
            pltpu.VMEM((LM_PER_TILE,), jnp.int32),
            pltpu.VMEM((LM_SUB, LM_DIM), f32),
        ],
    )
    def lm_gather(bank_hbm, idx_hbm, out_hbm, idxv, rows):
        k = lax.axis_index("c")
        s = lax.axis_index("s")
        base = (k * NS + s) * LM_PER_TILE
        pltpu.sync_copy(idx_hbm.at[pl.ds(base, LM_PER_TILE)], idxv)
        for b in range(LM_PER_TILE // LM_SUB):
            pltpu.sync_copy(bank_hbm.at[idxv.at[pl.ds(b * LM_SUB, LM_SUB)]],
                            rows)
            pltpu.sync_copy(rows, out_hbm.at[pl.ds(base + b * LM_SUB, LM_SUB)])

    return lm_gather


BLK = 1000
GRID = N // BLK


def _tc1_body(nt_ref, sh_ref, temb_ref, tacp_ref, wr_ref, x0c_ref, y0r_ref):
    nt = nt_ref[...]
    sh = sh_ref[...]
    t_type = jnp.zeros((BLK, 32), f32)
    for kk in range(3):
        t_type = t_type + (nt == kk).astype(f32) * temb_ref[pl.ds(kk, 1), :]
    onehot = (sh == lax.broadcasted_iota(jnp.int32, (1, 640), 1)).astype(f32)
    t_tac = jnp.dot(onehot, tacp_ref[...], preferred_element_type=f32)
    x0p = jnp.concatenate(
        [t_type, t_tac, jnp.ones((BLK, 1), f32), jnp.zeros((BLK, 31), f32)],
        axis=1)
    x0c_ref[0, :, :] = x0p[:, :CW]
    x0c_ref[1, :, :] = x0p[:, CW:]
    y0r_ref[...] = jnp.dot(x0p, wr_ref[...], preferred_element_type=f32)


def _tc2_body(p_ref, y0r_ref, wl_ref, bl_ref, x1c_ref, invd_ref):
    p = p_ref[...]
    ssum = jnp.concatenate([p[0], p[1]], axis=1)
    deg = ssum[:, 96:97]
    invd = 1.0 / jnp.maximum(deg, 1.0)
    mean1 = ssum * invd
    x1 = jnp.maximum(
        jnp.dot(mean1, wl_ref[...], preferred_element_type=f32)
        + bl_ref[...] + y0r_ref[...], 0.0)
    for c in range(8):
        x1c_ref[c, :, :] = x1[:, c * CW:(c + 1) * CW]
    invd_ref[...] = invd


def _tc3_body(a2_ref, x1c_ref, invd_ref, batch_ref, wl_ref, bl_ref, wr_ref,
              gw1_ref, gb1_ref, gw2_ref, gb2_ref, x2_ref, g_ref, gmax_ref):
    i = pl.program_id(0)
    invd = invd_ref[...]
    acc = jnp.broadcast_to(bl_ref[...], (BLK, HID))
    for c in range(8):
        acc = acc + _dot16(a2_ref[c] * invd, wl_ref[pl.ds(c * CW, CW), :])
        acc = acc + _dot16(x1c_ref[c], wr_ref[pl.ds(c * CW, CW), :])
    x2 = jnp.maximum(acc, 0.0)
    gh = jnp.maximum(_dot16(x2, gw1_ref[...]) + gb1_ref[...], 0.0)
    g = jnp.dot(gh, gw2_ref[...], preferred_element_type=f32) + gb2_ref[...]
    x2_ref[...] = x2
    g_ref[...] = g
    bm = batch_ref[...] == lax.broadcasted_iota(jnp.int32, (1, B), 1)
    cand = jnp.where(bm, g, -1e30)
    blockmax = jnp.max(cand, axis=0, keepdims=True)

    @pl.when(i == 0)
    def _():
        gmax_ref[...] = jnp.full((1, B), -1e30, f32)

    gmax_ref[...] = jnp.maximum(gmax_ref[...], blockmax)


def _tc4_body(x2_ref, g_ref, batch_ref, gmax_ref, lm_ref, sid_ref,
              sw_ref, sb_ref, lng_ref, lnb_ref,
              w1a_ref, w1b_ref, b1_ref, w2_ref, b2_ref,
              out_ref, S_ref, d_ref, Hs_ref, cnt_ref):
    i = pl.program_id(0)

    @pl.when(i == 0)
    def _():
        S_ref[...] = jnp.zeros((B, HID), f32)
        d_ref[...] = jnp.zeros((B, 1), f32)
        Hs_ref[...] = jnp.zeros((B, STATE_DIM), f32)
        cnt_ref[...] = jnp.zeros((B, 1), f32)

    P = (batch_ref[...] == lax.broadcasted_iota(jnp.int32, (1, B), 1)
         ).astype(f32)
    gmaxsel = jnp.sum(P * gmax_ref[...], axis=1, keepdims=True)
    ex = jnp.exp(g_ref[...] - gmaxsel)
    dn = (((0,), (0,)), ((), ()))
    S_ref[...] = S_ref[...] + lax.dot_general(
        P.astype(bf16), (ex * x2_ref[...]).astype(bf16), dn,
        preferred_element_type=f32)
    d_ref[...] = d_ref[...] + lax.dot_general(
        P, ex, dn, preferred_element_type=f32)

    hb = jnp.maximum(
        _dot16(lm_ref[...], sw_ref[...]) + sb_ref[...], 0.0)
    mu = jnp.mean(hb, axis=1, keepdims=True)
    var = jnp.mean((hb - mu) * (hb - mu), axis=1, keepdims=True)
    h = (hb - mu) / jnp.sqrt(var + 1e-5) * lng_ref[...] + lnb_ref[...]
    mask = (sid_ref[...] >= 0).astype(f32)
    h = h * mask
    Hs_ref[...] = Hs_ref[...] + lax.dot_general(
        P.astype(bf16), h.astype(bf16), dn, preferred_element_type=f32)
    cnt_ref[...] = cnt_ref[...] + lax.dot_general(
        P, mask, dn, preferred_element_type=f32)

    @pl.when(i == GRID - 1)
    def _():
        graph_struct = S_ref[...] / (d_ref[...] + 1e-16)
        state_sem = Hs_ref[...] / (cnt_ref[...] + 1e-6)
        hcls = jnp.maximum(
            jnp.dot(graph_struct, w1a_ref[...], preferred_element_type=f32)
            + jnp.dot(state_sem, w1b_ref[...], preferred_element_type=f32)
            + b1_ref[...], 0.0)
        out_ref[...] = (jnp.dot(hcls, w2_ref[...], preferred_element_type=f32)
                        + b2_ref[...])


def _full(shape):
    return pl.BlockSpec(shape, lambda i: (0,) * len(shape))


bf16 = jnp.bfloat16


def _dot16(a, b):
    return jnp.dot(a.astype(bf16), b.astype(bf16), preferred_element_type=f32)


def kernel(node_type, node_tactic_id, state_lm_id, batch, edge_index,
           state_lm_bank, params):
    p = params
    src = edge_index[0]
    dst = edge_index[1]
    npad = E_PAD - E
    pad_src = jnp.arange(npad, dtype=jnp.int32) % N
    pad_dst = N + (jnp.arange(npad, dtype=jnp.int32) % (N_ACC - N))
    src_p = jnp.concatenate([src, pad_src])
    dst_p = jnp.concatenate([dst, pad_dst])
    dstb = dst_p.reshape(NBLK, 2, EB // 2)
    src2 = (src_p[None, :]
            + (jnp.arange(2, dtype=jnp.int32) * N)[:, None]).reshape(
                2 * NBLK, 2, EB // 2)
    src8 = (src_p[None, :]
            + (jnp.arange(8, dtype=jnp.int32) * N)[:, None]).reshape(
                8 * NBLK, 2, EB // 2)

    nt2 = node_type[:, None]
    sh2 = jnp.clip(node_tactic_id + 1, 0, NUM_TACTICS)[:, None]
    batch2 = batch[:, None]
    sid2 = state_lm_id[:, None]

    temb = jnp.zeros((8, 32), f32).at[:3].set(p["type_emb"])
    tacp = jnp.zeros((640, 64), f32).at[:NUM_TACTICS + 1].set(p["tactic_emb"])
    wr_pad = jnp.zeros((128, HID), f32).at[:96].set(p["c1_Wr"])
    wl_pad = jnp.zeros((128, HID), f32).at[:96].set(p["c1_Wl"])
    bl1 = p["c1_bl"][None, :]
    bl2 = p["c2_bl"][None, :]
    gb1 = p["gate_b1"][None, :]
    gb2 = p["gate_b2"][None, :]
    sb = p["state_b"][None, :]
    lng = p["state_ln_g"][None, :]
    lnb = p["state_ln_b"][None, :]
    w1a = p["cls_W1"][:HID]
    w1b = p["cls_W1"][HID:]
    b1 = p["cls_b1"][None, :]
    b2 = p["cls_b2"][None, :]

    lm_idx = jnp.concatenate([
        jnp.clip(state_lm_id, 0),
        jnp.arange(N_LM_PAD - N, dtype=jnp.int32) % 17])

    x0c, y0r = pl.pallas_call(
        _tc1_body,
        grid=(GRID,),
        in_specs=[
            pl.BlockSpec((BLK, 1), lambda i: (i, 0)),
            pl.BlockSpec((BLK, 1), lambda i: (i, 0)),
            _full((8, 32)),
            _full((640, 64)),
            _full((128, HID)),
        ],
        out_specs=[
            pl.BlockSpec((2, BLK, CW), lambda i: (0, i, 0)),
            pl.BlockSpec((BLK, HID), lambda i: (i, 0)),
        ],
        out_shape=[
            jax.ShapeDtypeStruct((2, N, CW), f32),
            jax.ShapeDtypeStruct((N, HID), f32),
        ],
    )(nt2, sh2, temb, tacp, wr_pad)

    agg1 = _make_segsum(2)(x0c.reshape(2 * N, CW), src2,
                           dstb).reshape(2, N_ACC, CW)

    lm = _make_lm_gather()(state_lm_bank, lm_idx)

    x1c, invd = pl.pallas_call(
        _tc2_body,
        grid=(GRID,),
        in_specs=[
            pl.BlockSpec((2, BLK, CW), lambda i: (0, i, 0)),
            pl.BlockSpec((BLK, HID), lambda i: (i, 0)),
            _full((128, HID)),
            _full((1, HID)),
        ],
        out_specs=[
            pl.BlockSpec((8, BLK, CW), lambda i: (0, i, 0)),
            pl.BlockSpec((BLK, 1), lambda i: (i, 0)),
        ],
        out_shape=[
            jax.ShapeDtypeStruct((8, N, CW), f32),
            jax.ShapeDtypeStruct((N, 1), f32),
        ],
    )(agg1, y0r, wl_pad, bl1)

    agg2 = _make_segsum(8)(x1c.reshape(8 * N, CW), src8,
                           dstb).reshape(8, N_ACC, CW)

    x2, g, gmax = pl.pallas_call(
        _tc3_body,
        grid=(GRID,),
        in_specs=[
            pl.BlockSpec((8, BLK, CW), lambda i: (0, i, 0)),
            pl.BlockSpec((8, BLK, CW), lambda i: (0, i, 0)),
            pl.BlockSpec((BLK, 1), lambda i: (i, 0)),
            pl.BlockSpec((BLK, 1), lambda i: (i, 0)),
            _full((HID, HID)),
            _full((1, HID)),
            _full((HID, HID)),
            _full((HID, HID)),
            _full((1, HID)),
            _full((HID, 1)),
            _full((1, 1)),
        ],
        out_specs=[
            pl.BlockSpec((BLK, HID), lambda i: (i, 0)),
            pl.BlockSpec((BLK, 1), lambda i: (i, 0)),
            pl.BlockSpec((1, B), lambda i: (0, 0)),
        ],
        out_shape=[
            jax.ShapeDtypeStruct((N, HID), f32),
            jax.ShapeDtypeStruct((N, 1), f32),
            jax.ShapeDtypeStruct((1, B), f32),
        ],
    )(agg2, x1c, invd, batch2, p["c2_Wl"], bl2, p["c2_Wr"],
      p["gate_W1"], gb1, p["gate_W2"], gb2)

    logits = pl.pallas_call(
        _tc4_body,
        grid=(GRID,),
        in_specs=[
            pl.BlockSpec((BLK, HID), lambda i: (i, 0)),
            pl.BlockSpec((BLK, 1), lambda i: (i, 0)),
            pl.BlockSpec((BLK, 1), lambda i: (i, 0)),
            _full((1, B)),
            pl.BlockSpec((BLK, LM_DIM), lambda i: (i, 0)),
            pl.BlockSpec((BLK, 1), lambda i: (i, 0)),
            _full((LM_DIM, STATE_DIM)),
            _full((1, STATE_DIM)),
            _full((1, STATE_DIM)),
            _full((1, STATE_DIM)),
            _full((HID, HID)),
            _full((STATE_DIM, HID)),
            _full((1, HID)),
            _full((HID, HID)),
            _full((1, HID)),
        ],
        out_specs=pl.BlockSpec((B, HID), lambda i: (0, 0)),
        out_shape=jax.ShapeDtypeStruct((B, HID), f32),
        scratch_shapes=[
            pltpu.VMEM((B, HID), f32),
            pltpu.VMEM((B, 1), f32),
            pltpu.VMEM((B, STATE_DIM), f32),
            pltpu.VMEM((B, 1), f32),
        ],
    )(x2, g, batch2, gmax, lm, sid2,
      p["state_W"], sb, lng, lnb, w1a, w1b, b1, p["cls_W2"], b2)

    return logits
